# baseline (speedup 1.0000x reference)
import functools

import jax
import jax.numpy as jnp
from jax import lax
from jax.experimental import pallas as pl
from jax.experimental.pallas import tpu as pltpu
from jax.experimental.pallas import tpu_sc as plsc

N = 10000
D = 128
E = 320000

NC = 2
NS = 16
NT = NC * NS
L = 16

NPAD = 10112
RPT = NPAD // NS
C = 200
NB = 5
EPT = E // NT
NFULL = EPT // C
DCOL = 80

@functools.cache
def _mesh():
    return plsc.VectorSubcoreMesh(core_axis_name="c", subcore_axis_name="s",
                                  num_cores=NC, num_subcores=NS)


def _wid():
    return lax.axis_index("c") * NS + lax.axis_index("s")


def _deg_body(adj_hbm, out_hbm, dbuf, deg_v):
    wid = _wid()
    zeros = jnp.zeros((L,), jnp.float32)
    ones = jnp.ones((L,), jnp.float32)

    def zero(i, _):
        deg_v[pl.ds(i * L, L)] = zeros
        return _

    lax.fori_loop(0, DCOL * 128 // L, zero, None)

    pltpu.sync_copy(adj_hbm.at[1, pl.ds(wid * EPT, EPT)], dbuf)

    def count(j, _):
        idx = dbuf[pl.ds(j * L, L)]
        plsc.addupdate_scatter(deg_v, [idx], ones)
        return _

    lax.fori_loop(0, EPT // L, count, None)
    pltpu.sync_copy(deg_v, out_hbm.at[wid])


@functools.cache
def _deg():
    return pl.kernel(
        _deg_body,
        out_type=jax.ShapeDtypeStruct((NT, DCOL * 128), jnp.float32),
        mesh=_mesh(),
        scratch_types=[
            pltpu.VMEM((EPT,), jnp.int32),
            pltpu.VMEM((DCOL * 128,), jnp.float32),
        ],
        compiler_params=pltpu.CompilerParams(needs_layout_passes=False, use_tc_tiling_on_sc=False),
    )


def _agg_body(adj_hbm, tab_hbm, out_hbm, sbuf, dbuf, rows, tab_sh, acc,
              gsem, ssem, isem, *, wf):
    c = lax.axis_index("c")
    s = lax.axis_index("s")
    wid = c * NS + s
    zeros = jnp.zeros((L,), jnp.float32)
    ebase = wid * EPT

    sd = pltpu.async_copy(adj_hbm.at[0, pl.ds(ebase, EPT)], sbuf, isem)
    dd = pltpu.async_copy(adj_hbm.at[1, pl.ds(ebase, EPT)], dbuf, isem)
    td = pltpu.async_copy(tab_hbm.at[pl.ds(s * RPT, RPT)],
                          tab_sh.at[pl.ds(s * RPT, RPT)], isem)

    def zrow(i, _):
        for j in range(wf // L):
            rows[0, i, pl.ds(j * L, L)] = zeros
        return _

    lax.fori_loop(0, C, zrow, None)
    base = s * RPT
    done = 0
    while done < RPT:
        n = min(C, RPT - done)
        pltpu.sync_copy(rows.at[0, pl.ds(0, n)], acc.at[pl.ds(base + done, n)])
        done += n
    sd.wait()
    dd.wait()
    td.wait()
    plsc.subcore_barrier()

    def step(k, _):
        g0 = k * NB
        gd = []
        for b in range(NB):
            idx = sbuf.at[pl.ds((g0 + b) * C, C)]
            gd.append(pltpu.async_copy(tab_sh.at[idx], rows.at[b],
                                       gsem.at[b]))
        wd = []
        for b in range(NB):
            gd[b].wait()
            didx = dbuf.at[pl.ds((g0 + b) * C, C)]
            wd.append(pltpu.async_copy(rows.at[b], acc.at[didx],
                                       ssem.at[b], add=True))
        for b in range(NB):
            wd[b].wait()
        return _

    lax.fori_loop(0, NFULL // NB, step, None)
    plsc.subcore_barrier()
    pltpu.sync_copy(acc.at[pl.ds(s * RPT, RPT)],
                    out_hbm.at[c, pl.ds(s * RPT, RPT)])


@functools.cache
def _make_agg(wf):
    return pl.kernel(
        functools.partial(_agg_body, wf=wf),
        out_type=jax.ShapeDtypeStruct((NC, NPAD, wf), jnp.float32),
        mesh=_mesh(),
        scratch_types=[
            pltpu.VMEM((EPT,), jnp.int32),
            pltpu.VMEM((EPT,), jnp.int32),
            pltpu.VMEM((NB, C, wf), jnp.float32),
            pltpu.VMEM_SHARED((NPAD, wf), jnp.float32),
            pltpu.VMEM_SHARED((NPAD, wf), jnp.float32),
            pltpu.SemaphoreType.DMA((NB,)),
            pltpu.SemaphoreType.DMA((NB,)),
            pltpu.SemaphoreType.DMA,
        ],
        compiler_params=pltpu.CompilerParams(needs_layout_passes=False, use_tc_tiling_on_sc=False),
    )


P4 = NPAD // 4
P8 = NPAD // 8
NR = N // 4
PROW = N // 8
NCOL = NPAD // 128


def _f32(shape):
    return jax.ShapeDtypeStruct(shape, jnp.float32)


def _tc0_body(feat4, bdw1, h1p):
    h1p[pl.ds(0, NR), :] = jnp.dot(feat4[...], bdw1[...],
                                   preferred_element_type=jnp.float32)
    h1p[pl.ds(NR, P4 - NR), :] = jnp.zeros((P4 - NR, 128), jnp.float32)


def _tc1_body(h1p, degp, m8, m4, scaled1p, d8, d4):
    deg = degp[pl.ds(0, DCOL), :]
    for p in range(1, NT):
        deg = deg + degp[pl.ds(p * DCOL, DCOL), :]
    dinv = lax.rsqrt(lax.slice(deg, (0, 0), (NCOL, 128)) + 1.0)
    d8v = jnp.dot(dinv, m8[...],
                  preferred_element_type=jnp.float32).reshape(P8, 128)
    d4v = jnp.dot(dinv, m4[...],
                  preferred_element_type=jnp.float32).reshape(P4, 128)
    d8[...] = d8v
    d4[...] = d4v
    scaled1p[...] = h1p[...] * d4v


def _ln_relu_p(x, avg, g, b):
    m = jnp.dot(x, avg, preferred_element_type=jnp.float32)
    v = jnp.dot((x - m) ** 2, avg, preferred_element_type=jnp.float32)
    return jnp.maximum((x - m) * lax.rsqrt(v + 1e-5) * g + b, 0.0)


def _tc_l1_body(accp, scaled1p, d4, d8, b, g, be, a32, bdw2, scaled2p):
    conv = (accp[0] + accp[1] + scaled1p[...]) * d4[...] + b[...]
    a = _ln_relu_p(conv, a32[...], g[...], be[...])
    h = jnp.dot(a.reshape(P8, 256), bdw2[...],
                preferred_element_type=jnp.float32)
    scaled2p[...] = h * d8[...]


def _tc_l2_body(accp, scaled2p, d8, b, g, be, a16, bdw3, act2p, scaled3p):
    conv = (accp[0] + accp[1] + scaled2p[...]) * d8[...] + b[...]
    a = _ln_relu_p(conv, a16[...], g[...], be[...])
    act2p[...] = a
    scaled3p[...] = jnp.dot(a, bdw3[...],
                            preferred_element_type=jnp.float32) * d8[...]


def _tc_fin_body(accp, scaled3p, d8, b, g, be, a16, act2p, fw1, fb1, fw2,
                 fb2, logits):
    conv = (accp[0] + accp[1] + scaled3p[...]) * d8[...] + b[...]
    h = _ln_relu_p(conv, a16[...], g[...], be[...]) + act2p[...]
    mask = lax.broadcasted_iota(jnp.int32, (P8, 1), 0) < PROW
    s = jnp.sum(jnp.where(mask, h, 0.0), axis=0, keepdims=True)
    mx = jnp.max(jnp.where(mask, h, -3.0e38), axis=0, keepdims=True)
    mean = sum(s[:, 16 * k:16 * (k + 1)] for k in range(8)) / N
    mxf = mx[:, 0:16]
    for k in range(1, 8):
        mxf = jnp.maximum(mxf, mx[:, 16 * k:16 * (k + 1)])
    rep = jnp.concatenate([mean, mxf], axis=1)
    o = jnp.maximum(jnp.dot(rep, fw1[...],
                            preferred_element_type=jnp.float32) + fb1[...], 0.0)
    logits[...] = jnp.dot(o, fw2[...],
                          preferred_element_type=jnp.float32) + fb2[...]


_tc0 = pl.pallas_call(_tc0_body, out_shape=_f32((P4, 128)))
_tc1 = pl.pallas_call(
    _tc1_body,
    out_shape=(_f32((P4, 128)), _f32((P8, 128)), _f32((P4, 128))))
_tc_l1 = pl.pallas_call(_tc_l1_body, out_shape=_f32((P8, 128)))
_tc_l2 = pl.pallas_call(_tc_l2_body,
                        out_shape=(_f32((P8, 128)), _f32((P8, 128))))
_tc_fin = pl.pallas_call(_tc_fin_body, out_shape=_f32((1, 2)))


def _sel_matrix(k):
    cols = k * 128
    q = jnp.arange(cols) // 128
    l = jnp.arange(cols) % 128
    node = k * q + l // (128 // k)
    return (jnp.arange(128)[:, None] == node[None, :]).astype(jnp.float32)


@jax.jit
def kernel(adj, features, W1, b1, g1, be1, W2, b2, g2, be2, W3, b3, g3, be3,
           fw1, fb1, fw2, fb2):
    eye4 = jnp.eye(4, dtype=jnp.float32)
    eye8 = jnp.eye(8, dtype=jnp.float32)
    bdw1 = jnp.kron(eye4, W1)
    bdw2 = jnp.kron(eye8, W2)
    bdw3 = jnp.kron(eye8, W3)
    a32 = jnp.kron(eye4, jnp.full((32, 32), 1 / 32, jnp.float32))
    a16 = jnp.kron(eye8, jnp.full((16, 16), 1 / 16, jnp.float32))

    q8 = jnp.arange(16 * 128) // 128
    l8 = jnp.arange(16 * 128) % 128
    node8 = 8 * q8 + l8 // 16
    m8 = (jnp.arange(128)[:, None] == node8[None, :]).astype(jnp.float32)
    q4 = jnp.arange(32 * 128) // 128
    l4 = jnp.arange(32 * 128) % 128
    node4 = 4 * q4 + l4 // 32
    m4 = (jnp.arange(128)[:, None] == node4[None, :]).astype(jnp.float32)

    feat4 = features.reshape(NR, 512)

    degp = _deg()(adj)
    h1p = _tc0(feat4, bdw1)
    scaled1p, d8, d4 = _tc1(h1p, degp.reshape(NT * DCOL, 128), m8, m4)
    acc1 = _make_agg(32)(adj, scaled1p.reshape(NPAD, 32))
    scaled2p = _tc_l1(acc1.reshape(2, P4, 128), scaled1p, d4, d8,
                      jnp.tile(b1, 4)[None], jnp.tile(g1, 4)[None],
                      jnp.tile(be1, 4)[None], a32, bdw2)
    acc2 = _make_agg(16)(adj, scaled2p.reshape(NPAD, 16))
    act2p, scaled3p = _tc_l2(acc2.reshape(2, P8, 128), scaled2p, d8,
                             jnp.tile(b2, 8)[None], jnp.tile(g2, 8)[None],
                             jnp.tile(be2, 8)[None], a16, bdw3)
    acc3 = _make_agg(16)(adj, scaled3p.reshape(NPAD, 16))
    return _tc_fin(acc3.reshape(2, P8, 128), scaled3p, d8,
                   jnp.tile(b3, 8)[None], jnp.tile(g3, 8)[None],
                   jnp.tile(be3, 8)[None], a16, act2p,
                   fw1, fb1[None], fw2, fb2[None])


# speedup vs baseline: 1.0612x; 1.0612x over previous
import functools

import jax
import jax.numpy as jnp
from jax import lax
from jax.experimental import pallas as pl
from jax.experimental.pallas import tpu as pltpu
from jax.experimental.pallas import tpu_sc as plsc

N = 10000
D = 128
E = 320000

NC = 2
NS = 16
NT = NC * NS
L = 16

NPAD = 10112
RPT = NPAD // NS
C = 128
NB = 6
EPT = E // NT
NFULL = EPT // C
TAIL = EPT - NFULL * C
DCOL = 80

@functools.cache
def _mesh():
    return plsc.VectorSubcoreMesh(core_axis_name="c", subcore_axis_name="s",
                                  num_cores=NC, num_subcores=NS)


def _wid():
    return lax.axis_index("c") * NS + lax.axis_index("s")


def _deg_body(adj_hbm, out_hbm, dbuf, deg_v):
    wid = _wid()
    zeros = jnp.zeros((L,), jnp.float32)
    ones = jnp.ones((L,), jnp.float32)

    def zero(i, _):
        deg_v[pl.ds(i * L, L)] = zeros
        return _

    lax.fori_loop(0, DCOL * 128 // L, zero, None)

    pltpu.sync_copy(adj_hbm.at[1, pl.ds(wid * EPT, EPT)], dbuf)

    def count(j, _):
        idx = dbuf[pl.ds(j * L, L)]
        plsc.addupdate_scatter(deg_v, [idx], ones)
        return _

    lax.fori_loop(0, EPT // L, count, None)
    pltpu.sync_copy(deg_v, out_hbm.at[wid])


@functools.cache
def _deg():
    return pl.kernel(
        _deg_body,
        out_type=jax.ShapeDtypeStruct((NT, DCOL * 128), jnp.float32),
        mesh=_mesh(),
        scratch_types=[
            pltpu.VMEM((EPT,), jnp.int32),
            pltpu.VMEM((DCOL * 128,), jnp.float32),
        ],
        compiler_params=pltpu.CompilerParams(needs_layout_passes=False, use_tc_tiling_on_sc=False),
    )


def _agg_body(adj_hbm, tab_hbm, out_hbm, sbuf, dbuf, rows, tab_sh, acc,
              gsem, ssem, isem, *, wf):
    c = lax.axis_index("c")
    s = lax.axis_index("s")
    wid = c * NS + s
    zeros = jnp.zeros((L,), jnp.float32)
    ebase = wid * EPT

    sd = pltpu.async_copy(adj_hbm.at[0, pl.ds(ebase, EPT)], sbuf, isem)
    dd = pltpu.async_copy(adj_hbm.at[1, pl.ds(ebase, EPT)], dbuf, isem)
    td = pltpu.async_copy(tab_hbm.at[pl.ds(s * RPT, RPT)],
                          tab_sh.at[pl.ds(s * RPT, RPT)], isem)

    def zrow(i, _):
        for j in range(wf // L):
            rows[0, i, pl.ds(j * L, L)] = zeros
        return _

    lax.fori_loop(0, C, zrow, None)
    base = s * RPT
    done = 0
    while done < RPT:
        n = min(C, RPT - done)
        pltpu.sync_copy(rows.at[0, pl.ds(0, n)], acc.at[pl.ds(base + done, n)])
        done += n
    sd.wait()
    dd.wait()
    td.wait()
    plsc.subcore_barrier()

    def step(k, _):
        g0 = k * NB
        gd = []
        for b in range(NB):
            idx = sbuf.at[pl.ds((g0 + b) * C, C)]
            gd.append(pltpu.async_copy(tab_sh.at[idx], rows.at[b],
                                       gsem.at[b]))
        wd = []
        for b in range(NB):
            gd[b].wait()
            didx = dbuf.at[pl.ds((g0 + b) * C, C)]
            wd.append(pltpu.async_copy(rows.at[b], acc.at[didx],
                                       ssem.at[b], add=True))
        for b in range(NB):
            wd[b].wait()
        return _

    lax.fori_loop(0, NFULL // NB, step, None)

    tidx = sbuf.at[pl.ds(NFULL * C, TAIL)]
    pltpu.async_copy(tab_sh.at[tidx], rows.at[0, pl.ds(0, TAIL)],
                     gsem.at[0]).wait()
    tdidx = dbuf.at[pl.ds(NFULL * C, TAIL)]
    pltpu.async_copy(rows.at[0, pl.ds(0, TAIL)], acc.at[tdidx],
                     ssem.at[0], add=True).wait()
    plsc.subcore_barrier()
    pltpu.sync_copy(acc.at[pl.ds(s * RPT, RPT)],
                    out_hbm.at[c, pl.ds(s * RPT, RPT)])


@functools.cache
def _make_agg(wf):
    return pl.kernel(
        functools.partial(_agg_body, wf=wf),
        out_type=jax.ShapeDtypeStruct((NC, NPAD, wf), jnp.float32),
        mesh=_mesh(),
        scratch_types=[
            pltpu.VMEM((EPT,), jnp.int32),
            pltpu.VMEM((EPT,), jnp.int32),
            pltpu.VMEM((NB, C, wf), jnp.float32),
            pltpu.VMEM_SHARED((NPAD, wf), jnp.float32),
            pltpu.VMEM_SHARED((NPAD, wf), jnp.float32),
            pltpu.SemaphoreType.DMA((NB,)),
            pltpu.SemaphoreType.DMA((NB,)),
            pltpu.SemaphoreType.DMA,
        ],
        compiler_params=pltpu.CompilerParams(needs_layout_passes=False, use_tc_tiling_on_sc=False),
    )


P4 = NPAD // 4
P8 = NPAD // 8
NR = N // 4
PROW = N // 8
NCOL = NPAD // 128


def _f32(shape):
    return jax.ShapeDtypeStruct(shape, jnp.float32)


def _tc0_body(feat4, bdw1, h1p):
    h1p[pl.ds(0, NR), :] = jnp.dot(feat4[...], bdw1[...],
                                   preferred_element_type=jnp.float32)
    h1p[pl.ds(NR, P4 - NR), :] = jnp.zeros((P4 - NR, 128), jnp.float32)


def _tc1_body(h1p, degp, m8, m4, scaled1p, d8, d4):
    deg = degp[pl.ds(0, DCOL), :]
    for p in range(1, NT):
        deg = deg + degp[pl.ds(p * DCOL, DCOL), :]
    dinv = lax.rsqrt(lax.slice(deg, (0, 0), (NCOL, 128)) + 1.0)
    d8v = jnp.dot(dinv, m8[...],
                  preferred_element_type=jnp.float32).reshape(P8, 128)
    d4v = jnp.dot(dinv, m4[...],
                  preferred_element_type=jnp.float32).reshape(P4, 128)
    d8[...] = d8v
    d4[...] = d4v
    scaled1p[...] = h1p[...] * d4v


def _ln_relu_p(x, avg, g, b):
    m = jnp.dot(x, avg, preferred_element_type=jnp.float32)
    v = jnp.dot((x - m) ** 2, avg, preferred_element_type=jnp.float32)
    return jnp.maximum((x - m) * lax.rsqrt(v + 1e-5) * g + b, 0.0)


def _tc_l1_body(accp, scaled1p, d4, d8, b, g, be, a32, bdw2, scaled2p):
    conv = (accp[0] + accp[1] + scaled1p[...]) * d4[...] + b[...]
    a = _ln_relu_p(conv, a32[...], g[...], be[...])
    h = jnp.dot(a.reshape(P8, 256), bdw2[...],
                preferred_element_type=jnp.float32)
    scaled2p[...] = h * d8[...]


def _tc_l2_body(accp, scaled2p, d8, b, g, be, a16, bdw3, act2p, scaled3p):
    conv = (accp[0] + accp[1] + scaled2p[...]) * d8[...] + b[...]
    a = _ln_relu_p(conv, a16[...], g[...], be[...])
    act2p[...] = a
    scaled3p[...] = jnp.dot(a, bdw3[...],
                            preferred_element_type=jnp.float32) * d8[...]


def _tc_fin_body(accp, scaled3p, d8, b, g, be, a16, act2p, fw1, fb1, fw2,
                 fb2, logits):
    conv = (accp[0] + accp[1] + scaled3p[...]) * d8[...] + b[...]
    h = _ln_relu_p(conv, a16[...], g[...], be[...]) + act2p[...]
    mask = lax.broadcasted_iota(jnp.int32, (P8, 1), 0) < PROW
    s = jnp.sum(jnp.where(mask, h, 0.0), axis=0, keepdims=True)
    mx = jnp.max(jnp.where(mask, h, -3.0e38), axis=0, keepdims=True)
    mean = sum(s[:, 16 * k:16 * (k + 1)] for k in range(8)) / N
    mxf = mx[:, 0:16]
    for k in range(1, 8):
        mxf = jnp.maximum(mxf, mx[:, 16 * k:16 * (k + 1)])
    rep = jnp.concatenate([mean, mxf], axis=1)
    o = jnp.maximum(jnp.dot(rep, fw1[...],
                            preferred_element_type=jnp.float32) + fb1[...], 0.0)
    logits[...] = jnp.dot(o, fw2[...],
                          preferred_element_type=jnp.float32) + fb2[...]


_tc0 = pl.pallas_call(_tc0_body, out_shape=_f32((P4, 128)))
_tc1 = pl.pallas_call(
    _tc1_body,
    out_shape=(_f32((P4, 128)), _f32((P8, 128)), _f32((P4, 128))))
_tc_l1 = pl.pallas_call(_tc_l1_body, out_shape=_f32((P8, 128)))
_tc_l2 = pl.pallas_call(_tc_l2_body,
                        out_shape=(_f32((P8, 128)), _f32((P8, 128))))
_tc_fin = pl.pallas_call(_tc_fin_body, out_shape=_f32((1, 2)))


def _sel_matrix(k):
    cols = k * 128
    q = jnp.arange(cols) // 128
    l = jnp.arange(cols) % 128
    node = k * q + l // (128 // k)
    return (jnp.arange(128)[:, None] == node[None, :]).astype(jnp.float32)


@jax.jit
def kernel(adj, features, W1, b1, g1, be1, W2, b2, g2, be2, W3, b3, g3, be3,
           fw1, fb1, fw2, fb2):
    eye4 = jnp.eye(4, dtype=jnp.float32)
    eye8 = jnp.eye(8, dtype=jnp.float32)
    bdw1 = jnp.kron(eye4, W1)
    bdw2 = jnp.kron(eye8, W2)
    bdw3 = jnp.kron(eye8, W3)
    a32 = jnp.kron(eye4, jnp.full((32, 32), 1 / 32, jnp.float32))
    a16 = jnp.kron(eye8, jnp.full((16, 16), 1 / 16, jnp.float32))

    q8 = jnp.arange(16 * 128) // 128
    l8 = jnp.arange(16 * 128) % 128
    node8 = 8 * q8 + l8 // 16
    m8 = (jnp.arange(128)[:, None] == node8[None, :]).astype(jnp.float32)
    q4 = jnp.arange(32 * 128) // 128
    l4 = jnp.arange(32 * 128) % 128
    node4 = 4 * q4 + l4 // 32
    m4 = (jnp.arange(128)[:, None] == node4[None, :]).astype(jnp.float32)

    feat4 = features.reshape(NR, 512)

    degp = _deg()(adj)
    h1p = _tc0(feat4, bdw1)
    scaled1p, d8, d4 = _tc1(h1p, degp.reshape(NT * DCOL, 128), m8, m4)
    acc1 = _make_agg(32)(adj, scaled1p.reshape(NPAD, 32))
    scaled2p = _tc_l1(acc1.reshape(2, P4, 128), scaled1p, d4, d8,
                      jnp.tile(b1, 4)[None], jnp.tile(g1, 4)[None],
                      jnp.tile(be1, 4)[None], a32, bdw2)
    acc2 = _make_agg(16)(adj, scaled2p.reshape(NPAD, 16))
    act2p, scaled3p = _tc_l2(acc2.reshape(2, P8, 128), scaled2p, d8,
                             jnp.tile(b2, 8)[None], jnp.tile(g2, 8)[None],
                             jnp.tile(be2, 8)[None], a16, bdw3)
    acc3 = _make_agg(16)(adj, scaled3p.reshape(NPAD, 16))
    return _tc_fin(acc3.reshape(2, P8, 128), scaled3p, d8,
                   jnp.tile(b3, 8)[None], jnp.tile(g3, 8)[None],
                   jnp.tile(be3, 8)[None], a16, act2p,
                   fw1, fb1[None], fw2, fb2[None])


# speedup vs baseline: 1.0664x; 1.0049x over previous
import functools

import jax
import jax.numpy as jnp
from jax import lax
from jax.experimental import pallas as pl
from jax.experimental.pallas import tpu as pltpu
from jax.experimental.pallas import tpu_sc as plsc

N = 10000
D = 128
E = 320000

NC = 2
NS = 16
NT = NC * NS
L = 16

NPAD = 10112
RPT = NPAD // NS
C = 128
NB = 6
EPT = E // NT
NFULL = EPT // C
TAIL = EPT - NFULL * C
DCOL = 80

@functools.cache
def _mesh():
    return plsc.VectorSubcoreMesh(core_axis_name="c", subcore_axis_name="s",
                                  num_cores=NC, num_subcores=NS)


def _wid():
    return lax.axis_index("c") * NS + lax.axis_index("s")


def _deg_body(adj_hbm, out_hbm, dbuf, deg_v, deg_w):
    wid = _wid()
    zeros = jnp.zeros((L,), jnp.float32)
    ones = jnp.ones((L,), jnp.float32)

    def zero(i, _):
        deg_v[pl.ds(i * L, L)] = zeros
        deg_w[pl.ds(i * L, L)] = zeros
        return _

    lax.fori_loop(0, DCOL * 128 // L, zero, None)

    pltpu.sync_copy(adj_hbm.at[1, pl.ds(wid * EPT, EPT)], dbuf)

    def count(j, _):
        idx = dbuf[pl.ds(2 * j * L, L)]
        idx2 = dbuf[pl.ds((2 * j + 1) * L, L)]
        plsc.addupdate_scatter(deg_v, [idx], ones)
        plsc.addupdate_scatter(deg_w, [idx2], ones)
        return _

    lax.fori_loop(0, EPT // (2 * L), count, None)

    def merge(i, _):
        deg_v[pl.ds(i * L, L)] = (deg_v[pl.ds(i * L, L)]
                                  + deg_w[pl.ds(i * L, L)])
        return _

    lax.fori_loop(0, DCOL * 128 // L, merge, None)
    pltpu.sync_copy(deg_v, out_hbm.at[wid])


@functools.cache
def _deg():
    return pl.kernel(
        _deg_body,
        out_type=jax.ShapeDtypeStruct((NT, DCOL * 128), jnp.float32),
        mesh=_mesh(),
        scratch_types=[
            pltpu.VMEM((EPT,), jnp.int32),
            pltpu.VMEM((DCOL * 128,), jnp.float32),
            pltpu.VMEM((DCOL * 128,), jnp.float32),
        ],
        compiler_params=pltpu.CompilerParams(needs_layout_passes=False, use_tc_tiling_on_sc=False),
    )


def _agg_body(adj_hbm, tab_hbm, out_hbm, sbuf, dbuf, rows, tab_sh, acc,
              gsem, ssem, isem, *, wf):
    c = lax.axis_index("c")
    s = lax.axis_index("s")
    wid = c * NS + s
    zeros = jnp.zeros((L,), jnp.float32)
    ebase = wid * EPT

    sd = pltpu.async_copy(adj_hbm.at[0, pl.ds(ebase, EPT)], sbuf, isem)
    dd = pltpu.async_copy(adj_hbm.at[1, pl.ds(ebase, EPT)], dbuf, isem)
    td = pltpu.async_copy(tab_hbm.at[pl.ds(s * RPT, RPT)],
                          tab_sh.at[pl.ds(s * RPT, RPT)], isem)

    def zrow(i, _):
        for j in range(wf // L):
            rows[0, i, pl.ds(j * L, L)] = zeros
        return _

    lax.fori_loop(0, C, zrow, None)
    base = s * RPT
    done = 0
    while done < RPT:
        n = min(C, RPT - done)
        pltpu.sync_copy(rows.at[0, pl.ds(0, n)], acc.at[pl.ds(base + done, n)])
        done += n
    sd.wait()
    dd.wait()
    td.wait()
    plsc.subcore_barrier()

    def step(k, _):
        g0 = k * NB
        gd = []
        for b in range(NB):
            idx = sbuf.at[pl.ds((g0 + b) * C, C)]
            gd.append(pltpu.async_copy(tab_sh.at[idx], rows.at[b],
                                       gsem.at[b]))
        wd = []
        for b in range(NB):
            gd[b].wait()
            didx = dbuf.at[pl.ds((g0 + b) * C, C)]
            wd.append(pltpu.async_copy(rows.at[b], acc.at[didx],
                                       ssem.at[b], add=True))
        for b in range(NB):
            wd[b].wait()
        return _

    lax.fori_loop(0, NFULL // NB, step, None)

    tidx = sbuf.at[pl.ds(NFULL * C, TAIL)]
    pltpu.async_copy(tab_sh.at[tidx], rows.at[0, pl.ds(0, TAIL)],
                     gsem.at[0]).wait()
    tdidx = dbuf.at[pl.ds(NFULL * C, TAIL)]
    pltpu.async_copy(rows.at[0, pl.ds(0, TAIL)], acc.at[tdidx],
                     ssem.at[0], add=True).wait()
    plsc.subcore_barrier()
    pltpu.sync_copy(acc.at[pl.ds(s * RPT, RPT)],
                    out_hbm.at[c, pl.ds(s * RPT, RPT)])


@functools.cache
def _make_agg(wf):
    return pl.kernel(
        functools.partial(_agg_body, wf=wf),
        out_type=jax.ShapeDtypeStruct((NC, NPAD, wf), jnp.float32),
        mesh=_mesh(),
        scratch_types=[
            pltpu.VMEM((EPT,), jnp.int32),
            pltpu.VMEM((EPT,), jnp.int32),
            pltpu.VMEM((NB, C, wf), jnp.float32),
            pltpu.VMEM_SHARED((NPAD, wf), jnp.float32),
            pltpu.VMEM_SHARED((NPAD, wf), jnp.float32),
            pltpu.SemaphoreType.DMA((NB,)),
            pltpu.SemaphoreType.DMA((NB,)),
            pltpu.SemaphoreType.DMA,
        ],
        compiler_params=pltpu.CompilerParams(needs_layout_passes=False, use_tc_tiling_on_sc=False),
    )


P4 = NPAD // 4
P8 = NPAD // 8
NR = N // 4
PROW = N // 8
NCOL = NPAD // 128


def _f32(shape):
    return jax.ShapeDtypeStruct(shape, jnp.float32)


def _tc0_body(feat4, bdw1, h1p):
    h1p[pl.ds(0, NR), :] = jnp.dot(feat4[...], bdw1[...],
                                   preferred_element_type=jnp.float32)
    h1p[pl.ds(NR, P4 - NR), :] = jnp.zeros((P4 - NR, 128), jnp.float32)


def _tc1_body(h1p, degp, m4, scaled1p, d4, dv):
    deg = degp[pl.ds(0, DCOL), :]
    for p in range(1, NT):
        deg = deg + degp[pl.ds(p * DCOL, DCOL), :]
    dinv = lax.rsqrt(lax.slice(deg, (0, 0), (NCOL, 128)) + 1.0)
    dv[...] = dinv
    d4v = jnp.dot(dinv, m4[...],
                  preferred_element_type=jnp.float32).reshape(P4, 128)
    d4[...] = d4v
    scaled1p[...] = h1p[...] * d4v


def _tc1b_body(dv, m8, d8):
    d8[...] = jnp.dot(dv[...], m8[...],
                      preferred_element_type=jnp.float32).reshape(P8, 128)


def _ln_relu_p(x, avg, g, b):
    m = jnp.dot(x, avg, preferred_element_type=jnp.float32)
    v = jnp.dot((x - m) ** 2, avg, preferred_element_type=jnp.float32)
    return jnp.maximum((x - m) * lax.rsqrt(v + 1e-5) * g + b, 0.0)


def _tc_l1_body(accp, scaled1p, d4, d8, b, g, be, a32, bdw2, scaled2p):
    conv = (accp[0] + accp[1] + scaled1p[...]) * d4[...] + b[...]
    a = _ln_relu_p(conv, a32[...], g[...], be[...])
    h = jnp.dot(a.reshape(P8, 256), bdw2[...],
                preferred_element_type=jnp.float32)
    scaled2p[...] = h * d8[...]


def _tc_l2_body(accp, scaled2p, d8, b, g, be, a16, bdw3, act2p, scaled3p):
    conv = (accp[0] + accp[1] + scaled2p[...]) * d8[...] + b[...]
    a = _ln_relu_p(conv, a16[...], g[...], be[...])
    act2p[...] = a
    scaled3p[...] = jnp.dot(a, bdw3[...],
                            preferred_element_type=jnp.float32) * d8[...]


def _tc_fin_body(accp, scaled3p, d8, b, g, be, a16, act2p, fw1, fb1, fw2,
                 fb2, logits):
    conv = (accp[0] + accp[1] + scaled3p[...]) * d8[...] + b[...]
    h = _ln_relu_p(conv, a16[...], g[...], be[...]) + act2p[...]
    mask = lax.broadcasted_iota(jnp.int32, (P8, 1), 0) < PROW
    s = jnp.sum(jnp.where(mask, h, 0.0), axis=0, keepdims=True)
    mx = jnp.max(jnp.where(mask, h, -3.0e38), axis=0, keepdims=True)
    mean = sum(s[:, 16 * k:16 * (k + 1)] for k in range(8)) / N
    mxf = mx[:, 0:16]
    for k in range(1, 8):
        mxf = jnp.maximum(mxf, mx[:, 16 * k:16 * (k + 1)])
    rep = jnp.concatenate([mean, mxf], axis=1)
    o = jnp.maximum(jnp.dot(rep, fw1[...],
                            preferred_element_type=jnp.float32) + fb1[...], 0.0)
    logits[...] = jnp.dot(o, fw2[...],
                          preferred_element_type=jnp.float32) + fb2[...]


_tc0 = pl.pallas_call(_tc0_body, out_shape=_f32((P4, 128)))
_tc1 = pl.pallas_call(
    _tc1_body,
    out_shape=(_f32((P4, 128)), _f32((P4, 128)), _f32((NCOL, 128))))
_tc1b = pl.pallas_call(_tc1b_body, out_shape=_f32((P8, 128)))
_tc_l1 = pl.pallas_call(_tc_l1_body, out_shape=_f32((P8, 128)))
_tc_l2 = pl.pallas_call(_tc_l2_body,
                        out_shape=(_f32((P8, 128)), _f32((P8, 128))))
_tc_fin = pl.pallas_call(_tc_fin_body, out_shape=_f32((1, 2)))


def _sel_matrix(k):
    cols = k * 128
    q = jnp.arange(cols) // 128
    l = jnp.arange(cols) % 128
    node = k * q + l // (128 // k)
    return (jnp.arange(128)[:, None] == node[None, :]).astype(jnp.float32)


@jax.jit
def kernel(adj, features, W1, b1, g1, be1, W2, b2, g2, be2, W3, b3, g3, be3,
           fw1, fb1, fw2, fb2):
    eye4 = jnp.eye(4, dtype=jnp.float32)
    eye8 = jnp.eye(8, dtype=jnp.float32)
    bdw1 = jnp.kron(eye4, W1)
    bdw2 = jnp.kron(eye8, W2)
    bdw3 = jnp.kron(eye8, W3)
    a32 = jnp.kron(eye4, jnp.full((32, 32), 1 / 32, jnp.float32))
    a16 = jnp.kron(eye8, jnp.full((16, 16), 1 / 16, jnp.float32))

    q8 = jnp.arange(16 * 128) // 128
    l8 = jnp.arange(16 * 128) % 128
    node8 = 8 * q8 + l8 // 16
    m8 = (jnp.arange(128)[:, None] == node8[None, :]).astype(jnp.float32)
    q4 = jnp.arange(32 * 128) // 128
    l4 = jnp.arange(32 * 128) % 128
    node4 = 4 * q4 + l4 // 32
    m4 = (jnp.arange(128)[:, None] == node4[None, :]).astype(jnp.float32)

    feat4 = features.reshape(NR, 512)

    degp = _deg()(adj)
    h1p = _tc0(feat4, bdw1)
    scaled1p, d4, dv = _tc1(h1p, degp.reshape(NT * DCOL, 128), m4)
    d8 = _tc1b(dv, m8)
    acc1 = _make_agg(32)(adj, scaled1p.reshape(NPAD, 32))
    scaled2p = _tc_l1(acc1.reshape(2, P4, 128), scaled1p, d4, d8,
                      jnp.tile(b1, 4)[None], jnp.tile(g1, 4)[None],
                      jnp.tile(be1, 4)[None], a32, bdw2)
    acc2 = _make_agg(16)(adj, scaled2p.reshape(NPAD, 16))
    act2p, scaled3p = _tc_l2(acc2.reshape(2, P8, 128), scaled2p, d8,
                             jnp.tile(b2, 8)[None], jnp.tile(g2, 8)[None],
                             jnp.tile(be2, 8)[None], a16, bdw3)
    acc3 = _make_agg(16)(adj, scaled3p.reshape(NPAD, 16))
    return _tc_fin(acc3.reshape(2, P8, 128), scaled3p, d8,
                   jnp.tile(b3, 8)[None], jnp.tile(g3, 8)[None],
                   jnp.tile(be3, 8)[None], a16, act2p,
                   fw1, fb1[None], fw2, fb2[None])
